# baseline (speedup 1.0000x reference)
import functools

import jax
import jax.numpy as jnp
from jax import lax
from jax.experimental import pallas as pl
from jax.experimental.pallas import tpu as pltpu
from jax.experimental.pallas import tpu_sc as plsc

_NC = 2
_NS = 16
_NW = _NC * _NS
_L = 16
_LN2 = 0.6931471805599453
_SQRT2 = 1.4142135623730951


def _log_f32(p):
    bits = plsc.bitcast(p, jnp.int32)
    e = (bits >> 23) - 127
    m = plsc.bitcast((bits & 0x7FFFFF) | 0x3F800000, jnp.float32)
    big = m > _SQRT2
    m = jnp.where(big, m * 0.5, m)
    e = jnp.where(big, e + 1, e)
    s = (m - 1.0) / (m + 1.0)
    s2 = s * s
    ln_m = 2.0 * s * (1.0 + s2 * (1.0 / 3.0 + s2 * (1.0 / 5.0 + s2 * (1.0 / 7.0 + s2 * (1.0 / 9.0)))))
    ln_p = ln_m + e.astype(jnp.float32) * _LN2
    return jnp.where(p == 0.0, -jnp.inf, ln_p)


def _sc_loss_partials(in_t, tgt1d):
    C, N = in_t.shape
    CW = N // _NW
    W = 128
    NQ = CW // W
    mesh = plsc.VectorSubcoreMesh(
        core_axis_name="c", subcore_axis_name="s",
        num_cores=_NC, num_subcores=_NS,
    )

    @functools.partial(
        pl.kernel,
        out_type=jax.ShapeDtypeStruct((_NW * _L,), jnp.float32),
        mesh=mesh,
        scratch_types=[
            pltpu.VMEM((CW,), jnp.int32),
            pltpu.VMEM((_L,), jnp.float32),
            pltpu.VMEM((NQ, W, W), jnp.float32),
            pltpu.SemaphoreType.DMA((4,)),
        ],
        compiler_params=pltpu.CompilerParams(
            use_tc_tiling_on_sc=True, needs_layout_passes=False,
            skip_device_barrier=True,
        ),
    )
    def loss_kernel(in_hbm, tgt_hbm, out_hbm, tgt_v, acc_v, win_v, sem):
        wid = lax.axis_index("s") * _NC + lax.axis_index("c")
        col0 = wid * CW
        pltpu.sync_copy(tgt_hbm.at[pl.ds(col0, CW)], tgt_v)
        copies = []
        for q in range(NQ):
            rows = tgt_v.at[pl.ds(q * W, W)]
            copies.append(pltpu.async_copy(
                in_hbm.at[rows, pl.ds(col0 + q * W, W)], win_v.at[q],
                sem.at[q % 4]))
        lane = lax.iota(jnp.int32, _L)
        acc = jnp.zeros((_L,), jnp.float32)
        for q in range(NQ):
            copies[q].wait()
            qv = jnp.full((_L,), q, jnp.int32)

            def body(g, a, qv=qv):
                d = g * _L + lane
                p = plsc.load_gather(win_v, [qv, d, d])
                r = 1.0 - p
                return a + (_log_f32(p) - r * r)

            acc = lax.fori_loop(0, W // _L, body, acc)
        acc_v[...] = acc
        pltpu.sync_copy(acc_v, out_hbm.at[pl.ds(wid * _L, _L)])

    return loss_kernel(in_t, tgt1d)


def _tc_mean(partials2d, n):
    def body(p_ref, o_ref):
        o_ref[0, 0] = jnp.sum(p_ref[...]) * (1.0 / n)

    return pl.pallas_call(
        body,
        out_shape=jax.ShapeDtypeStruct((1, 1), jnp.float32),
        out_specs=pl.BlockSpec(memory_space=pltpu.SMEM),
        compiler_params=pltpu.CompilerParams(skip_device_barrier=True),
    )(partials2d)


def kernel(inputs, targets):
    N, C = inputs.shape
    tgt1d = targets.astype(jnp.int32).reshape(N)
    part = _sc_loss_partials(inputs.T, tgt1d)
    return _tc_mean(part.reshape(4, 128), N)[0, 0]


# speedup vs baseline: 1.0118x; 1.0118x over previous
import functools

import jax
import jax.numpy as jnp
from jax import lax
from jax.experimental import pallas as pl
from jax.experimental.pallas import tpu as pltpu
from jax.experimental.pallas import tpu_sc as plsc

_NC = 2
_NS = 16
_NW = _NC * _NS
_L = 16
_LN2 = 0.6931471805599453
_SQRT2 = 1.4142135623730951


def _log_f32(p):
    bits = plsc.bitcast(p, jnp.int32)
    e = (bits >> 23) - 127
    m = plsc.bitcast((bits & 0x7FFFFF) | 0x3F800000, jnp.float32)
    big = m > _SQRT2
    m = jnp.where(big, m * 0.5, m)
    e = jnp.where(big, e + 1, e)
    s = (m - 1.0) / (m + 1.0)
    s2 = s * s
    ln_m = 2.0 * s * (1.0 + s2 * (1.0 / 3.0 + s2 * (1.0 / 5.0 + s2 * (1.0 / 7.0 + s2 * (1.0 / 9.0)))))
    ln_p = ln_m + e.astype(jnp.float32) * _LN2
    return jnp.where(p == 0.0, -jnp.inf, ln_p)


def _sc_loss_partials(in_t, tgt1d):
    C, N = in_t.shape
    CW = N // _NW
    W = 128
    NQ = CW // W
    mesh = plsc.VectorSubcoreMesh(
        core_axis_name="c", subcore_axis_name="s",
        num_cores=_NC, num_subcores=_NS,
    )

    @functools.partial(
        pl.kernel,
        out_type=jax.ShapeDtypeStruct((_NW * _L,), jnp.float32),
        mesh=mesh,
        scratch_types=[
            pltpu.VMEM((CW,), jnp.int32),
            pltpu.VMEM((_L,), jnp.float32),
            pltpu.VMEM((NQ, W, W), jnp.float32),
            pltpu.SemaphoreType.DMA,
        ],
        compiler_params=pltpu.CompilerParams(
            use_tc_tiling_on_sc=True, needs_layout_passes=False,
        ),
    )
    def loss_kernel(in_hbm, tgt_hbm, out_hbm, tgt_v, acc_v, win_v, sem):
        wid = lax.axis_index("s") * _NC + lax.axis_index("c")
        col0 = wid * CW
        pltpu.sync_copy(tgt_hbm.at[pl.ds(col0, CW)], tgt_v)
        copies = []
        for q in range(NQ):
            rows = tgt_v.at[pl.ds(q * W, W)]
            copies.append(pltpu.async_copy(
                in_hbm.at[rows, pl.ds(col0 + q * W, W)], win_v.at[q], sem))
        for cp in copies:
            cp.wait()
        lane = lax.iota(jnp.int32, _L)

        def body(i, acc):
            q = i >> 3
            d = (i & 7) * _L + lane
            p = plsc.load_gather(win_v, [jnp.full((_L,), 0, jnp.int32) + q, d, d])
            r = 1.0 - p
            return acc + (_log_f32(p) - r * r)

        acc = lax.fori_loop(0, NQ * (W // _L), body, jnp.zeros((_L,), jnp.float32))
        acc_v[...] = acc
        pltpu.sync_copy(acc_v, out_hbm.at[pl.ds(wid * _L, _L)])

    return loss_kernel(in_t, tgt1d)


def _tc_mean(partials2d, n):
    def body(p_ref, o_ref):
        o_ref[0, 0] = jnp.sum(p_ref[...]) * (1.0 / n)

    return pl.pallas_call(
        body,
        out_shape=jax.ShapeDtypeStruct((1, 1), jnp.float32),
        out_specs=pl.BlockSpec(memory_space=pltpu.SMEM),
    )(partials2d)


def kernel(inputs, targets):
    N, C = inputs.shape
    tgt1d = targets.astype(jnp.int32).reshape(N)
    part = _sc_loss_partials(inputs.T, tgt1d)
    return _tc_mean(part.reshape(4, 128), N)[0, 0]


# speedup vs baseline: 1.1067x; 1.0937x over previous
import functools

import jax
import jax.numpy as jnp
from jax import lax
from jax.experimental import pallas as pl
from jax.experimental.pallas import tpu as pltpu
from jax.experimental.pallas import tpu_sc as plsc

_NC = 2
_NS = 16
_NW = _NC * _NS
_L = 16
_LN2 = 0.6931471805599453
_SQRT2 = 1.4142135623730951


def _log_f32(p):
    bits = plsc.bitcast(p, jnp.int32)
    e = (bits >> 23) - 127
    m = plsc.bitcast((bits & 0x7FFFFF) | 0x3F800000, jnp.float32)
    big = m > _SQRT2
    m = jnp.where(big, m * 0.5, m)
    e = jnp.where(big, e + 1, e)
    s = (m - 1.0) / (m + 1.0)
    s2 = s * s
    ln_m = 2.0 * s * (1.0 + s2 * (1.0 / 3.0 + s2 * (1.0 / 5.0 + s2 * (1.0 / 7.0 + s2 * (1.0 / 9.0)))))
    ln_p = ln_m + e.astype(jnp.float32) * _LN2
    return jnp.where(p == 0.0, -jnp.inf, ln_p)


def _sc_loss_partials(in_t, tgt1d):
    C, N = in_t.shape
    CW = N // _NW
    W = 128
    NQ = CW // W
    mesh = plsc.VectorSubcoreMesh(
        core_axis_name="c", subcore_axis_name="s",
        num_cores=_NC, num_subcores=_NS,
    )

    @functools.partial(
        pl.kernel,
        out_type=jax.ShapeDtypeStruct((_NW * _L,), jnp.float32),
        mesh=mesh,
        scratch_types=[
            pltpu.VMEM((CW,), jnp.int32),
            pltpu.VMEM((_L,), jnp.float32),
            pltpu.VMEM((NQ, W, W), jnp.float32),
            pltpu.SemaphoreType.DMA,
        ],
        compiler_params=pltpu.CompilerParams(
            use_tc_tiling_on_sc=True, needs_layout_passes=False,
        ),
    )
    def loss_kernel(in_hbm, tgt_hbm, out_hbm, tgt_v, acc_v, win_v, sem):
        wid = lax.axis_index("s") * _NC + lax.axis_index("c")
        col0 = wid * CW
        pltpu.sync_copy(tgt_hbm.at[pl.ds(col0, CW)], tgt_v)
        lane = lax.iota(jnp.int32, _L)

        def window_src(q):
            rows = tgt_v.at[pl.ds(q * W, W)]
            return in_hbm.at[rows, pl.ds(col0 + q * W, W)]

        def fire(q, carry):
            pltpu.async_copy(window_src(q), win_v.at[q], sem)
            return carry

        lax.fori_loop(0, NQ, fire, 0)

        def body(i, acc):
            q = i >> 3
            d = (i & 7) * _L + lane
            p = plsc.load_gather(win_v, [jnp.full((_L,), 0, jnp.int32) + q, d, d])
            r = 1.0 - p
            return acc + (_log_f32(p) - r * r)

        def consume(q, acc):
            pltpu.make_async_copy(window_src(q), win_v.at[q], sem).wait()
            return lax.fori_loop(q * (W // _L), (q + 1) * (W // _L), body, acc)

        acc = lax.fori_loop(0, NQ, consume, jnp.zeros((_L,), jnp.float32))
        acc_v[...] = acc
        pltpu.sync_copy(acc_v, out_hbm.at[pl.ds(wid * _L, _L)])

    return loss_kernel(in_t, tgt1d)


def _tc_mean(partials2d, n):
    def body(p_ref, o_ref):
        o_ref[0, 0] = jnp.sum(p_ref[...]) * (1.0 / n)

    return pl.pallas_call(
        body,
        out_shape=jax.ShapeDtypeStruct((1, 1), jnp.float32),
        out_specs=pl.BlockSpec(memory_space=pltpu.SMEM),
    )(partials2d)


def kernel(inputs, targets):
    N, C = inputs.shape
    tgt1d = targets.astype(jnp.int32).reshape(N)
    part = _sc_loss_partials(inputs.T, tgt1d)
    return _tc_mean(part.reshape(4, 128), N)[0, 0]
